# baseline (speedup 1.0000x reference)
import functools

import jax
import jax.numpy as jnp
from jax import lax
from jax.experimental import pallas as pl
from jax.experimental.pallas import tpu as pltpu
from jax.experimental.pallas import tpu_sc as plsc

_R = 128
_N = 32768
_L = 16
_NCH = _N // _L
_NWORK = 32
_ROWS_PER = _R // _NWORK
_CAP = _N + _L
_U = 1
_UNROLL = 8


def _splat(x):
    return lax.broadcast(x, (_L,))


def _process_row(row_v, cand_v):
    lane = jnp.arange(_L, dtype=jnp.int32)
    zf = jnp.zeros((_L,), jnp.float32)
    zi = jnp.zeros((_L,), jnp.int32)
    onei = jnp.ones((_L,), jnp.int32)
    neg = jnp.full((_L,), -3.0e38, jnp.float32)

    lanebase = lane * _NCH
    @plsc.parallel_loop(0, _NCH, step=_U, unroll=_UNROLL,
                        carry=(lanebase, (neg,) * _U))
    def cpl(i, st):
        ptr_v, accs = st
        new_accs = []
        for u in range(_U):
            v = row_v[pl.ds((i + u) * _L, _L)]
            m = v > (accs[u] - 1.0)
            plsc.store_scatter(cand_v, [ptr_v], v, mask=m)
            ptr_v = ptr_v + jnp.where(m, onei, zi)
            new_accs.append(jnp.maximum(accs[u], v))
        return ptr_v, tuple(new_accs)

    ptr_v, accs = cpl
    acc = accs[0]
    for u in range(1, _U):
        acc = jnp.maximum(acc, accs[u])
    thr_v = _splat(jnp.max(acc)) - 1.0
    fill_v = thr_v - 1.0
    cnt_v = ptr_v - lanebase
    maxc = jnp.max(cnt_v)

    @plsc.parallel_loop(0, maxc, carry=None)
    def padl(j):
        plsc.store_scatter(cand_v, [lanebase + j], fill_v,
                           mask=(cnt_v <= j))

    def newton(tau_v):
        def nb(j, c2):
            s_acc, c_acc = c2
            v = plsc.load_gather(cand_v, [lanebase + j])
            m = v > tau_v
            return (s_acc + jnp.where(m, v, zf),
                    c_acc + jnp.where(m, v * 0.0 + 1.0, zf))
        s_acc, c_acc = lax.fori_loop(0, maxc, nb, (zf, zf))
        s_t = _splat(jnp.sum(s_acc))
        c_t = jnp.maximum(_splat(jnp.sum(c_acc)), 1.0)
        return (s_t - 1.0) / c_t

    def w_cond(st):
        tau_v, prev_v, it = st
        return jnp.logical_and(it < 32, jnp.any(tau_v != prev_v))

    def w_body(st):
        tau_v, _, it = st
        return (newton(tau_v), tau_v, it + 1)

    tau0 = newton(thr_v)
    tau_v, _, _ = lax.while_loop(w_cond, w_body, (tau0, thr_v, jnp.int32(0)))

    @plsc.parallel_loop(0, _NCH, step=_U, unroll=_UNROLL)
    def opl(i):
        for u in range(_U):
            v = row_v[pl.ds((i + u) * _L, _L)]
            row_v[pl.ds((i + u) * _L, _L)] = jnp.maximum(v - tau_v, 0.0)


def _body(x_hbm, out_hbm, row_a, row_b, cand_v, sem_ia, sem_ib, sem_oa, sem_ob):
    cid = lax.axis_index("c")
    sid = lax.axis_index("s")
    base = (sid * 2 + cid) * _ROWS_PER
    bufs = (row_a, row_b)
    isems = (sem_ia, sem_ib)
    osems = (sem_oa, sem_ob)

    h_in = [pltpu.async_copy(x_hbm.at[base], row_a, sem_ia)]
    h_out = [None, None]
    for r in range(_ROWS_PER):
        cur = bufs[r % 2]
        if r + 1 < _ROWS_PER:
            if h_out[(r + 1) % 2] is not None:
                h_out[(r + 1) % 2].wait()
                h_out[(r + 1) % 2] = None
            h_in.append(pltpu.async_copy(
                x_hbm.at[base + r + 1], bufs[(r + 1) % 2], isems[(r + 1) % 2]))
        h_in[r].wait()
        _process_row(cur, cand_v)
        h_out[r % 2] = pltpu.async_copy(cur, out_hbm.at[base + r], osems[r % 2])
    for h in h_out:
        if h is not None:
            h.wait()


@jax.jit
def _sparsemax(x):
    fn = pl.kernel(
        _body,
        out_type=jax.ShapeDtypeStruct((_R, _N), jnp.float32),
        mesh=plsc.VectorSubcoreMesh(core_axis_name="c", subcore_axis_name="s"),
        compiler_params=pltpu.CompilerParams(needs_layout_passes=False),
        scratch_types=[
            pltpu.VMEM((_N,), jnp.float32),
            pltpu.VMEM((_N,), jnp.float32),
            pltpu.VMEM((_CAP,), jnp.float32),
            pltpu.SemaphoreType.DMA,
            pltpu.SemaphoreType.DMA,
            pltpu.SemaphoreType.DMA,
            pltpu.SemaphoreType.DMA,
        ],
    )
    return fn(x)


def kernel(inputs):
    return _sparsemax(inputs)


# speedup vs baseline: 1.6224x; 1.6224x over previous
import jax
import jax.numpy as jnp
from jax import lax
from jax.experimental import pallas as pl
from jax.experimental.pallas import tpu as pltpu
from jax.experimental.pallas import tpu_sc as plsc

_R = 128
_N = 32768
_L = 16
_NCH = _N // _L
_NWORK = 32
_ROWS_PER = _R // _NWORK
_CAP = _N + _L
_U = 1
_UNROLL = 8


def _splat(x):
    return lax.broadcast(x, (_L,))


def _process_row(row_v, cand_v):
    lane = jnp.arange(_L, dtype=jnp.int32)
    zf = jnp.zeros((_L,), jnp.float32)
    zi = jnp.zeros((_L,), jnp.int32)
    onei = jnp.ones((_L,), jnp.int32)
    neg = jnp.full((_L,), -3.0e38, jnp.float32)

    @plsc.parallel_loop(0, _NCH, step=_U, unroll=_UNROLL,
                        carry=(zi - 1, (neg,) * _U))
    def cpl(i, st):
        ptr_b, accs = st
        new_accs = []
        for u in range(_U):
            v = row_v[pl.ds((i + u) * _L, _L)]
            m = v > (accs[u] - 1.0)
            incl = plsc.cumsum(onei, mask=m)
            plsc.store_scatter(cand_v, [ptr_b + incl], v, mask=m)
            ptr_b = ptr_b + plsc.all_reduce_population_count(m)
            new_accs.append(jnp.maximum(accs[u], v))
        return ptr_b, tuple(new_accs)

    ptr_b, accs = cpl
    acc = accs[0]
    for u in range(1, _U):
        acc = jnp.maximum(acc, accs[u])
    thr_v = _splat(jnp.max(acc)) - 1.0
    plsc.store_scatter(cand_v, [ptr_b + 1 + lane], thr_v - 1.0)
    nch2 = lax.shift_right_logical(jnp.max(ptr_b) + _L, 4)

    def newton(tau_v):
        def nb(i, c2):
            s_acc, c_acc = c2
            v = cand_v[pl.ds(i * _L, _L)]
            m = v > tau_v
            return (s_acc + jnp.where(m, v, zf),
                    c_acc + jnp.where(m, v * 0.0 + 1.0, zf))
        s_acc, c_acc = lax.fori_loop(0, nch2, nb, (zf, zf))
        s_t = _splat(jnp.sum(s_acc))
        c_t = jnp.maximum(_splat(jnp.sum(c_acc)), 1.0)
        return (s_t - 1.0) / c_t

    def w_cond(st):
        tau_v, prev_v, it = st
        return jnp.logical_and(it < 32, jnp.any(tau_v != prev_v))

    def w_body(st):
        tau_v, _, it = st
        return (newton(tau_v), tau_v, it + 1)

    tau0 = newton(thr_v)
    tau_v, _, _ = lax.while_loop(w_cond, w_body, (tau0, thr_v, jnp.int32(0)))

    @plsc.parallel_loop(0, _NCH, step=_U, unroll=_UNROLL)
    def opl(i):
        for u in range(_U):
            v = row_v[pl.ds((i + u) * _L, _L)]
            row_v[pl.ds((i + u) * _L, _L)] = jnp.maximum(v - tau_v, 0.0)


def _body(x_hbm, out_hbm, row_a, row_b, cand_v, sem_ia, sem_ib, sem_oa, sem_ob):
    cid = lax.axis_index("c")
    sid = lax.axis_index("s")
    base = (sid * 2 + cid) * _ROWS_PER
    bufs = (row_a, row_b)
    isems = (sem_ia, sem_ib)
    osems = (sem_oa, sem_ob)

    h_in = [pltpu.async_copy(x_hbm.at[base], row_a, sem_ia)]
    h_out = [None, None]
    for r in range(_ROWS_PER):
        cur = bufs[r % 2]
        if r + 1 < _ROWS_PER:
            if h_out[(r + 1) % 2] is not None:
                h_out[(r + 1) % 2].wait()
                h_out[(r + 1) % 2] = None
            h_in.append(pltpu.async_copy(
                x_hbm.at[base + r + 1], bufs[(r + 1) % 2], isems[(r + 1) % 2]))
        h_in[r].wait()
        _process_row(cur, cand_v)
        h_out[r % 2] = pltpu.async_copy(cur, out_hbm.at[base + r], osems[r % 2])
    for h in h_out:
        if h is not None:
            h.wait()


@jax.jit
def _sparsemax(x):
    fn = pl.kernel(
        _body,
        out_type=jax.ShapeDtypeStruct((_R, _N), jnp.float32),
        mesh=plsc.VectorSubcoreMesh(core_axis_name="c", subcore_axis_name="s"),
        compiler_params=pltpu.CompilerParams(needs_layout_passes=False),
        scratch_types=[
            pltpu.VMEM((_N,), jnp.float32),
            pltpu.VMEM((_N,), jnp.float32),
            pltpu.VMEM((_CAP,), jnp.float32),
            pltpu.SemaphoreType.DMA,
            pltpu.SemaphoreType.DMA,
            pltpu.SemaphoreType.DMA,
            pltpu.SemaphoreType.DMA,
        ],
    )
    return fn(x)


def kernel(inputs):
    return _sparsemax(inputs)
